# baseline (speedup 1.0000x reference)
import functools

import jax
import jax.numpy as jnp
import numpy as np
from jax import lax
from jax.experimental import pallas as pl
from jax.experimental.pallas import tpu as pltpu
from jax.experimental.pallas import tpu_sc as plsc

HID = 256
GH = 512
LN_EPS = 1e-5
N_REAL = 10000
NP_ = 10240
RNG = 320
NW = 32
E_REAL = 170000
ECHUNK = 1024
NCHUNK = 168
EPAD = ECHUNK * NCHUNK
CAP = 6144
CAPA = 6176
GRP = 32
NG = CAP // GRP
ACC_ROWS = 324
F32 = jnp.float32
I32 = jnp.int32


def _ln(x, g, b):
    m = x.mean(-1, keepdims=True)
    v = ((x - m) ** 2).mean(-1, keepdims=True)
    return (x - m) / jnp.sqrt(v + LN_EPS) * g + b


def _gelu(x):
    return x * 0.5 * (1.0 + lax.erf(x * np.float32(1.0 / np.sqrt(2.0))))


def _tc_head_a_body(x_ref, wn_ref, bn_ref, gn_ref, betan_ref, w_ref, am_ref,
                    ad_ref, ha_ref, hb_ref, as_ref, adr_ref):
    x = x_ref[...]
    t = jnp.dot(x, wn_ref[...], preferred_element_type=F32) + bn_ref[...]
    nf = _gelu(_ln(t, gn_ref[...], betan_ref[...]))
    h = jnp.dot(nf, w_ref[...], preferred_element_type=F32)
    ha_ref[...] = h[:, :HID]
    hb_ref[...] = h[:, HID:]
    as_ref[...] = jnp.dot(h, am_ref[...], preferred_element_type=F32)
    adr_ref[...] = jnp.dot(h, ad_ref[...], preferred_element_type=F32)


def _tc_head_b_body(agg_ref, bprev_ref, w_ref, am_ref, ad_ref,
                    ha_ref, hb_ref, as_ref, adr_ref):
    nf = jnp.maximum(agg_ref[...] + bprev_ref[...], 0.0)
    h = jnp.dot(nf, w_ref[...], preferred_element_type=F32)
    ha_ref[...] = h[:, :HID]
    hb_ref[...] = h[:, HID:]
    as_ref[...] = jnp.dot(h, am_ref[...], preferred_element_type=F32)
    adr_ref[...] = jnp.dot(h, ad_ref[...], preferred_element_type=F32)


_ROWS_BLK = 256
_NBLK = NP_ // _ROWS_BLK


def _full(shape):
    return pl.BlockSpec(shape, lambda *_: tuple(0 for _ in shape))


def _tc_head_a(xp, wn, bn, gn, betan, w0, am, ad):
    return pl.pallas_call(
        _tc_head_a_body,
        grid=(_NBLK,),
        in_specs=[
            pl.BlockSpec((_ROWS_BLK, HID), lambda i: (i, 0)),
            _full((HID, GH)), _full((1, GH)), _full((1, GH)), _full((1, GH)),
            _full((GH, GH)), _full((GH, 128)), _full((GH, 128)),
        ],
        out_specs=[
            pl.BlockSpec((_ROWS_BLK, HID), lambda i: (i, 0)),
            pl.BlockSpec((_ROWS_BLK, HID), lambda i: (i, 0)),
            pl.BlockSpec((_ROWS_BLK, 128), lambda i: (i, 0)),
            pl.BlockSpec((_ROWS_BLK, 128), lambda i: (i, 0)),
        ],
        out_shape=[
            jax.ShapeDtypeStruct((NP_, HID), F32),
            jax.ShapeDtypeStruct((NP_, HID), F32),
            jax.ShapeDtypeStruct((NP_, 128), F32),
            jax.ShapeDtypeStruct((NP_, 128), F32),
        ],
    )(xp, wn, bn, gn, betan, w0, am, ad)


def _tc_head_b(agg, bprev, w, am, ad):
    return pl.pallas_call(
        _tc_head_b_body,
        grid=(_NBLK,),
        in_specs=[
            pl.BlockSpec((_ROWS_BLK, GH), lambda i: (i, 0)),
            _full((1, GH)),
            _full((GH, GH)), _full((GH, 128)), _full((GH, 128)),
        ],
        out_specs=[
            pl.BlockSpec((_ROWS_BLK, HID), lambda i: (i, 0)),
            pl.BlockSpec((_ROWS_BLK, HID), lambda i: (i, 0)),
            pl.BlockSpec((_ROWS_BLK, 128), lambda i: (i, 0)),
            pl.BlockSpec((_ROWS_BLK, 128), lambda i: (i, 0)),
        ],
        out_shape=[
            jax.ShapeDtypeStruct((NP_, HID), F32),
            jax.ShapeDtypeStruct((NP_, HID), F32),
            jax.ShapeDtypeStruct((NP_, 128), F32),
            jax.ShapeDtypeStruct((NP_, 128), F32),
        ],
    )(agg, bprev, w, am, ad)


def _tc_pool_body(agg_ref, bidx_ref, b2_ref, wop_ref, bop_ref, gop_ref,
                  betaop_ref, wv_ref, bv_ref, wmo_ref, bmo_ref, ones_ref,
                  att_ref):
    nf = agg_ref[...]
    bi = bidx_ref[...]
    p = (bi == lax.broadcasted_iota(I32, (NP_, 8), 1)).astype(F32)
    pool = lax.dot_general(p, nf, (((0,), (0,)), ((), ())),
                           preferred_element_type=F32)
    cnt = lax.dot_general(p, ones_ref[...], (((0,), (0,)), ((), ())),
                          preferred_element_type=F32)
    gf = pool / jnp.maximum(cnt[:, :1], 1.0) + b2_ref[...]
    gf = _ln(jnp.dot(gf, wop_ref[...], preferred_element_type=F32)
             + bop_ref[...], gop_ref[...], betaop_ref[...])
    att = jnp.dot(gf, wv_ref[...], preferred_element_type=F32) + bv_ref[...]
    att = jnp.dot(att, wmo_ref[...], preferred_element_type=F32) + bmo_ref[...]
    att_ref[...] = att


def _tc_pool(agg2, bidx8, b2, wop, bop, gop, betaop, wv, bv, wmo, bmo, ones):
    return pl.pallas_call(
        _tc_pool_body,
        in_specs=[
            _full((NP_, GH)), _full((NP_, 8)), _full((1, GH)),
            _full((GH, HID)), _full((1, HID)), _full((1, HID)), _full((1, HID)),
            _full((HID, HID)), _full((1, HID)), _full((HID, HID)),
            _full((1, HID)), _full((NP_, 128)),
        ],
        out_specs=_full((8, HID)),
        out_shape=jax.ShapeDtypeStruct((8, HID), F32),
    )(agg2, bidx8, b2, wop, bop, gop, betaop, wv, bv, wmo, bmo, ones)


def _tc_gate_body(va_ref, ta_ref, att_ref, wg1_ref, bg1_ref, gg_ref,
                  betag_ref, wg2_ref, bg2_ref, out_ref):
    va = va_ref[0]
    ta = ta_ref[0]
    atb = jnp.broadcast_to(att_ref[0], (GH, HID))
    fi = jnp.concatenate([va, ta, atb], axis=-1)
    g = _gelu(_ln(jnp.dot(fi, wg1_ref[...], preferred_element_type=F32)
                  + bg1_ref[...], gg_ref[...], betag_ref[...]))
    s = jnp.dot(g, wg2_ref[...], preferred_element_type=F32) + bg2_ref[...]
    lane = lax.broadcasted_iota(I32, s.shape, 1)
    s = jnp.where(lane < 3, s, -1e30)
    s = s - jnp.max(s, axis=-1, keepdims=True)
    e = jnp.exp(s)
    gw = e / jnp.sum(e, axis=-1, keepdims=True)
    out_ref[0] = (gw[:, 0:1] * va + gw[:, 1:2] * ta + gw[:, 2:3] * atb)


def _tc_gate(va, ta, att83, wg1, bg1, gg, betag, wg2p, bg2p):
    return pl.pallas_call(
        _tc_gate_body,
        grid=(8,),
        in_specs=[
            pl.BlockSpec((1, GH, HID), lambda b: (b, 0, 0)),
            pl.BlockSpec((1, GH, HID), lambda b: (b, 0, 0)),
            pl.BlockSpec((1, 1, HID), lambda b: (b, 0, 0)),
            _full((3 * HID, HID)), _full((1, HID)), _full((1, HID)),
            _full((1, HID)), _full((HID, 128)), _full((1, 128)),
        ],
        out_specs=pl.BlockSpec((1, GH, HID), lambda b: (b, 0, 0)),
        out_shape=jax.ShapeDtypeStruct((8, GH, HID), F32),
    )(va, ta, att83, wg1, bg1, gg, betag, wg2p, bg2p)


_LANES = lax.iota(I32, 16) if False else None


def _splat_i32(v):
    return jnp.zeros((16,), I32) + v


def _lanes():
    return lax.iota(I32, 16)


def _zero_acc(acc):
    lanes = _lanes()

    def body(i, _):
        row = _splat_i32(i // 16)
        col = _splat_i32((i % 16) * 16) + lanes
        plsc.store_scatter(acc, [row, col], jnp.zeros((16,), F32))
        return 0

    lax.fori_loop(0, ACC_ROWS * 16, body, 0)


def _apply_inv_and_writeout(acc, den_v, agg_h, lo, half):
    lanes = _lanes()

    def body(r, _):
        iv = plsc.load_gather(den_v, [_splat_i32(r)])
        for t in range(16):
            col = _splat_i32(t * 16) + lanes
            row = _splat_i32(r)
            cur = plsc.load_gather(acc, [row, col])
            plsc.store_scatter(acc, [row, col], cur * iv)
        return 0

    lax.fori_loop(0, RNG, body, 0)
    pltpu.sync_copy(acc.at[pl.ds(0, RNG)],
                    agg_h.at[pl.ds(lo, RNG), pl.ds(half * HID, HID)])


def _accumulate_half(h_ref, bsrc, bdstl, asrc_v, adst_v, acc, stg, sem0, sem1):
    lanes = _lanes()

    def start(g, b, sem):
        pltpu.make_async_copy(
            h_ref.at[bsrc.at[pl.ds(g * GRP, GRP)]], stg.at[b], sem).start()

    def wait(b, sem):
        pltpu.make_async_copy(
            h_ref.at[bsrc.at[pl.ds(0, GRP)]], stg.at[b], sem).wait()

    def process(g, b):
        def ebody(e, _):
            sl = _splat_i32(g * GRP) + e
            sv = plsc.load_gather(bsrc, [sl])
            rl = plsc.load_gather(bdstl, [sl])
            av = plsc.load_gather(asrc_v, [sv])
            bv = plsc.load_gather(adst_v, [jnp.minimum(rl, RNG - 1)])
            x = av + bv
            xl = jnp.where(x >= 0.0, x, 0.2 * x)
            wv = jnp.exp(xl)
            for t in range(16):
                col = _splat_i32(t * 16) + lanes
                xr = stg[b, e, pl.ds(t * 16, 16)]
                plsc.addupdate_scatter(acc, [rl, col], xr * wv)
            return 0

        lax.fori_loop(0, GRP, ebody, 0)

    start(0, 0, sem0)

    def body(i, _):
        g0 = i * 2
        start(g0 + 1, 1, sem1)
        wait(0, sem0)
        process(g0, 0)
        start(g0 + 2, 0, sem0)
        wait(1, sem1)
        process(g0 + 1, 1)
        return 0

    lax.fori_loop(0, NG // 2, body, 0)
    wait(0, sem0)


def _sc_common_tail(ha, hb, agg_h, bsrc, bdstl, asrc_v, adst_v, den_v, acc,
                    stg, sem0, sem1, lo):
    lanes = _lanes()
    for i in range(RNG // 16):
        d = den_v[pl.ds(i * 16, 16)]
        den_v[pl.ds(i * 16, 16)] = 1.0 / (d + 1e-16)
    for half, h_ref in ((0, ha), (1, hb)):
        _zero_acc(acc)
        _accumulate_half(h_ref, bsrc, bdstl, asrc_v, adst_v, acc, stg,
                         sem0, sem1)
        _apply_inv_and_writeout(acc, den_v, agg_h, lo, half)


def _make_sc_layer0():
    mesh = plsc.VectorSubcoreMesh(core_axis_name="c", subcore_axis_name="s")

    @functools.partial(
        pl.kernel,
        mesh=mesh,
        compiler_params=pltpu.CompilerParams(needs_layout_passes=False),
        out_type=[
            jax.ShapeDtypeStruct((NP_, GH), F32),
            jax.ShapeDtypeStruct((NW, CAPA), I32),
            jax.ShapeDtypeStruct((NW, CAPA), I32),
            jax.ShapeDtypeStruct((NW, 16), I32),
        ],
        scratch_types=[
            pltpu.VMEM((ACC_ROWS, HID), F32),
            pltpu.VMEM((NP_,), F32),
            pltpu.VMEM((RNG,), F32),
            pltpu.VMEM((RNG,), F32),
            pltpu.VMEM((CAPA,), I32),
            pltpu.VMEM((CAPA,), I32),
            pltpu.VMEM((ECHUNK,), I32),
            pltpu.VMEM((ECHUNK,), I32),
            pltpu.VMEM((2, GRP, HID), F32),
            pltpu.VMEM((16,), I32),
            pltpu.SemaphoreType.DMA,
            pltpu.SemaphoreType.DMA,
        ],
    )
    def sc0(ha, hb, asrc_h, adst_h, src_h, dst_h,
            agg_h, bsrc_h, bdstl_h, cnt_h,
            acc, asrc_v, adst_v, den_v, bsrc, bdstl, es, ed, stg, cbuf,
            sem0, sem1):
        c = lax.axis_index("c")
        s = lax.axis_index("s")
        wid = s * 2 + c
        lo = wid * RNG
        lanes = _lanes()

        def init_bins(i, _):
            idx = _splat_i32(i * 16) + lanes
            plsc.store_scatter(bsrc, [idx], jnp.zeros((16,), I32))
            plsc.store_scatter(bdstl, [idx], _splat_i32(RNG))
            return 0

        lax.fori_loop(0, CAPA // 16, init_bins, 0)
        for i in range(RNG // 16):
            den_v[pl.ds(i * 16, 16)] = jnp.zeros((16,), F32)

        pltpu.sync_copy(asrc_h, asrc_v)
        pltpu.sync_copy(adst_h.at[pl.ds(lo, RNG)], adst_v)

        def chunk(ch, cnt_vec):
            pltpu.sync_copy(src_h.at[pl.ds(ch * ECHUNK, ECHUNK)], es)
            pltpu.sync_copy(dst_h.at[pl.ds(ch * ECHUNK, ECHUNK)], ed)

            def inner(k, cnt_vec):
                idx = _splat_i32(k * 16) + lanes
                sv = plsc.load_gather(es, [idx])
                dv = plsc.load_gather(ed, [idx])
                m = (dv >= lo) & (dv < lo + RNG)
                dl = jnp.clip(dv - lo, 0, RNG - 1)
                av = plsc.load_gather(asrc_v, [sv])
                bv = plsc.load_gather(adst_v, [dl])
                x = av + bv
                e = jnp.where(x >= 0.0, x, 0.2 * x)
                w = jnp.where(m, jnp.exp(e), 0.0)
                plsc.addupdate_scatter(den_v, [dl], w, mask=m)
                inc = plsc.cumsum(m.astype(I32))
                pos = jnp.minimum(cnt_vec + inc - 1, CAP - 1)
                plsc.store_scatter(bsrc, [pos], sv, mask=m)
                plsc.store_scatter(bdstl, [pos], dl, mask=m)
                return cnt_vec + plsc.all_reduce_population_count(m)

            return lax.fori_loop(0, ECHUNK // 16, inner, cnt_vec)

        cnt_vec = lax.fori_loop(0, NCHUNK, chunk, jnp.zeros((16,), I32))

        cbuf[...] = cnt_vec
        pltpu.sync_copy(bsrc, bsrc_h.at[wid])
        pltpu.sync_copy(bdstl, bdstl_h.at[wid])
        pltpu.sync_copy(cbuf, cnt_h.at[wid])

        _sc_common_tail(ha, hb, agg_h, bsrc, bdstl, asrc_v, adst_v, den_v,
                        acc, stg, sem0, sem1, lo)

    return sc0


def _make_sc_layer_n():
    mesh = plsc.VectorSubcoreMesh(core_axis_name="c", subcore_axis_name="s")

    @functools.partial(
        pl.kernel,
        mesh=mesh,
        compiler_params=pltpu.CompilerParams(needs_layout_passes=False),
        out_type=jax.ShapeDtypeStruct((NP_, GH), F32),
        scratch_types=[
            pltpu.VMEM((ACC_ROWS, HID), F32),
            pltpu.VMEM((NP_,), F32),
            pltpu.VMEM((RNG,), F32),
            pltpu.VMEM((RNG,), F32),
            pltpu.VMEM((CAPA,), I32),
            pltpu.VMEM((CAPA,), I32),
            pltpu.VMEM((2, GRP, HID), F32),
            pltpu.SemaphoreType.DMA,
            pltpu.SemaphoreType.DMA,
        ],
    )
    def scn(ha, hb, asrc_h, adst_h, bsrc_h, bdstl_h,
            agg_h,
            acc, asrc_v, adst_v, den_v, bsrc, bdstl, stg,
            sem0, sem1):
        c = lax.axis_index("c")
        s = lax.axis_index("s")
        wid = s * 2 + c
        lo = wid * RNG
        lanes = _lanes()

        for i in range(RNG // 16):
            den_v[pl.ds(i * 16, 16)] = jnp.zeros((16,), F32)

        pltpu.sync_copy(asrc_h, asrc_v)
        pltpu.sync_copy(adst_h.at[pl.ds(lo, RNG)], adst_v)
        pltpu.sync_copy(bsrc_h.at[wid], bsrc)
        pltpu.sync_copy(bdstl_h.at[wid], bdstl)

        def wbody(i, _):
            idx = _splat_i32(i * 16) + lanes
            sv = plsc.load_gather(bsrc, [idx])
            dl = plsc.load_gather(bdstl, [idx])
            valid = dl < RNG
            dlc = jnp.minimum(dl, RNG - 1)
            av = plsc.load_gather(asrc_v, [sv])
            bv = plsc.load_gather(adst_v, [dlc])
            x = av + bv
            e = jnp.where(x >= 0.0, x, 0.2 * x)
            w = jnp.where(valid, jnp.exp(e), 0.0)
            plsc.addupdate_scatter(den_v, [dlc], w, mask=valid)
            return 0

        lax.fori_loop(0, CAPA // 16, wbody, 0)

        _sc_common_tail(ha, hb, agg_h, bsrc, bdstl, asrc_v, adst_v, den_v,
                        acc, stg, sem0, sem1, lo)

    return scn


_sc_layer0 = _make_sc_layer0()
_sc_layer_n = _make_sc_layer_n()


def kernel(visual_features, text_features, graph_nodes, params, edge_index,
           batch_idx):
    p = params
    B = visual_features.shape[0]

    xp = jnp.concatenate(
        [graph_nodes, jnp.zeros((NP_ - N_REAL, HID), F32)], axis=0)
    loops = jnp.arange(N_REAL, dtype=I32)
    src = jnp.concatenate([edge_index[0].astype(I32), loops,
                           jnp.zeros((EPAD - E_REAL,), I32)])
    dst = jnp.concatenate([edge_index[1].astype(I32), loops,
                           jnp.full((EPAD - E_REAL,), NP_, I32)])
    bidx8 = jnp.broadcast_to(
        jnp.concatenate([batch_idx.astype(I32),
                         jnp.full((NP_ - N_REAL,), 8, I32)])[:, None],
        (NP_, 8))
    row = lambda v: v.reshape(1, -1)
    mat128 = lambda v: jnp.broadcast_to(v.reshape(-1, 1), (v.shape[-1], 128))
    ones = jnp.ones((NP_, 128), F32)
    wg2p = jnp.concatenate([p['Wg2'], jnp.zeros((HID, 125), F32)], axis=1)
    bg2p = jnp.concatenate([p['bg2'], jnp.zeros((125,), F32)]).reshape(1, -1)
    va = jnp.concatenate(
        [visual_features,
         jnp.zeros((B, GH - visual_features.shape[1], HID), F32)], axis=1)

    ha, hb, asr, adr = _tc_head_a(
        xp, p['Wn'], row(p['bn']), row(p['gn']), row(p['betan']), p['W0'],
        mat128(p['as0'][0, 0]), mat128(p['ad0'][0, 0]))
    agg, bsrc_h, bdstl_h, _cnt = _sc_layer0(
        ha, hb, asr[:, 0], adr[:, 0], src, dst)

    for i in (1, 2):
        ha, hb, asr, adr = _tc_head_b(
            agg, row(p['b%d' % (i - 1)]), p['W%d' % i],
            mat128(p['as%d' % i][0, 0]), mat128(p['ad%d' % i][0, 0]))
        agg = _sc_layer_n(ha, hb, asr[:, 0], adr[:, 0], bsrc_h, bdstl_h)

    att8 = _tc_pool(agg, bidx8, row(p['b2']), p['Wop'], row(p['bop']),
                    row(p['gop']), row(p['betaop']), p['Wv'], row(p['bv']),
                    p['Wmo'], row(p['bmo']), ones)

    out = _tc_gate(va, text_features, att8.reshape(8, 1, HID),
                   p['Wg1'], row(p['bg1']), row(p['gg']), row(p['betag']),
                   wg2p, bg2p)
    return out


# speedup vs baseline: 1.6425x; 1.6425x over previous
import functools

import jax
import jax.numpy as jnp
import numpy as np
from jax import lax
from jax.experimental import pallas as pl
from jax.experimental.pallas import tpu as pltpu
from jax.experimental.pallas import tpu_sc as plsc

HID = 256
GH = 512
LN_EPS = 1e-5
N_REAL = 10000
NP_ = 10240
RNG = 320
HRNG = 160
NW = 32
E_REAL = 170000
ECHUNK = 1024
NCHUNK = 168
CAP2 = 3136
GRP = 16
CAPA2 = CAP2 + GRP
BINSZ = 2 * CAPA2
NG2 = CAP2 // GRP
ACC_ROWS = 336
F32 = jnp.float32
I32 = jnp.int32
U32 = jnp.uint32


def _ln(x, g, b):
    m = x.mean(-1, keepdims=True)
    v = ((x - m) ** 2).mean(-1, keepdims=True)
    return (x - m) / jnp.sqrt(v + LN_EPS) * g + b


def _gelu(x):
    return x * 0.5 * (1.0 + lax.erf(x * np.float32(1.0 / np.sqrt(2.0))))


def _pack_bf16(h):
    u = lax.bitcast_convert_type(h, U32)
    b = (u + (((u >> 16) & 1) + 0x7FFF)) >> 16
    lo = b[:, :HID]
    hi = b[:, HID:]
    return lax.bitcast_convert_type(lo | (hi << 16), I32)


def _tc_head_a_body(x_ref, wn_ref, bn_ref, gn_ref, betan_ref, w_ref, am_ref,
                    ad_ref, hp_ref, as_ref, adr_ref):
    x = x_ref[...]
    t = jnp.dot(x, wn_ref[...], preferred_element_type=F32) + bn_ref[...]
    nf = _gelu(_ln(t, gn_ref[...], betan_ref[...]))
    h = jnp.dot(nf, w_ref[...], preferred_element_type=F32)
    hp_ref[...] = _pack_bf16(h)
    as_ref[...] = jnp.dot(h, am_ref[...], preferred_element_type=F32)
    adr_ref[...] = jnp.dot(h, ad_ref[...], preferred_element_type=F32)


def _tc_head_b_body(agg_ref, bprev_ref, w_ref, am_ref, ad_ref,
                    hp_ref, as_ref, adr_ref):
    nf = jnp.maximum(agg_ref[...] + bprev_ref[...], 0.0)
    h = jnp.dot(nf, w_ref[...], preferred_element_type=F32)
    hp_ref[...] = _pack_bf16(h)
    as_ref[...] = jnp.dot(h, am_ref[...], preferred_element_type=F32)
    adr_ref[...] = jnp.dot(h, ad_ref[...], preferred_element_type=F32)


_ROWS_BLK = 256
_NBLK = NP_ // _ROWS_BLK


def _full(shape):
    return pl.BlockSpec(shape, lambda *_: tuple(0 for _ in shape))


def _tc_head_a(xp, wn, bn, gn, betan, w0, am, ad):
    return pl.pallas_call(
        _tc_head_a_body,
        grid=(_NBLK,),
        in_specs=[
            pl.BlockSpec((_ROWS_BLK, HID), lambda i: (i, 0)),
            _full((HID, GH)), _full((1, GH)), _full((1, GH)), _full((1, GH)),
            _full((GH, GH)), _full((GH, 128)), _full((GH, 128)),
        ],
        out_specs=[
            pl.BlockSpec((_ROWS_BLK, HID), lambda i: (i, 0)),
            pl.BlockSpec((_ROWS_BLK, 128), lambda i: (i, 0)),
            pl.BlockSpec((_ROWS_BLK, 128), lambda i: (i, 0)),
        ],
        out_shape=[
            jax.ShapeDtypeStruct((NP_, HID), I32),
            jax.ShapeDtypeStruct((NP_, 128), F32),
            jax.ShapeDtypeStruct((NP_, 128), F32),
        ],
    )(xp, wn, bn, gn, betan, w0, am, ad)


def _tc_head_b(agg, bprev, w, am, ad):
    return pl.pallas_call(
        _tc_head_b_body,
        grid=(_NBLK,),
        in_specs=[
            pl.BlockSpec((_ROWS_BLK, GH), lambda i: (i, 0)),
            _full((1, GH)),
            _full((GH, GH)), _full((GH, 128)), _full((GH, 128)),
        ],
        out_specs=[
            pl.BlockSpec((_ROWS_BLK, HID), lambda i: (i, 0)),
            pl.BlockSpec((_ROWS_BLK, 128), lambda i: (i, 0)),
            pl.BlockSpec((_ROWS_BLK, 128), lambda i: (i, 0)),
        ],
        out_shape=[
            jax.ShapeDtypeStruct((NP_, HID), I32),
            jax.ShapeDtypeStruct((NP_, 128), F32),
            jax.ShapeDtypeStruct((NP_, 128), F32),
        ],
    )(agg, bprev, w, am, ad)


def _tc_pool_body(agg_ref, bidx_ref, b2_ref, wop_ref, bop_ref, gop_ref,
                  betaop_ref, wv_ref, bv_ref, wmo_ref, bmo_ref, ones_ref,
                  att_ref):
    nf = agg_ref[...]
    bi = bidx_ref[...]
    p = (bi == lax.broadcasted_iota(I32, (NP_, 8), 1)).astype(F32)
    pool = lax.dot_general(p, nf, (((0,), (0,)), ((), ())),
                           preferred_element_type=F32)
    cnt = lax.dot_general(p, ones_ref[...], (((0,), (0,)), ((), ())),
                          preferred_element_type=F32)
    gf = pool / jnp.maximum(cnt[:, :1], 1.0) + b2_ref[...]
    gf = _ln(jnp.dot(gf, wop_ref[...], preferred_element_type=F32)
             + bop_ref[...], gop_ref[...], betaop_ref[...])
    att = jnp.dot(gf, wv_ref[...], preferred_element_type=F32) + bv_ref[...]
    att = jnp.dot(att, wmo_ref[...], preferred_element_type=F32) + bmo_ref[...]
    att_ref[...] = att


def _tc_pool(agg2, bidx8, b2, wop, bop, gop, betaop, wv, bv, wmo, bmo, ones):
    return pl.pallas_call(
        _tc_pool_body,
        in_specs=[
            _full((NP_, GH)), _full((NP_, 8)), _full((1, GH)),
            _full((GH, HID)), _full((1, HID)), _full((1, HID)), _full((1, HID)),
            _full((HID, HID)), _full((1, HID)), _full((HID, HID)),
            _full((1, HID)), _full((NP_, 128)),
        ],
        out_specs=_full((8, HID)),
        out_shape=jax.ShapeDtypeStruct((8, HID), F32),
    )(agg2, bidx8, b2, wop, bop, gop, betaop, wv, bv, wmo, bmo, ones)


def _tc_gate_body(va_ref, ta_ref, att_ref, wg1_ref, bg1_ref, gg_ref,
                  betag_ref, wg2_ref, bg2_ref, out_ref):
    va = va_ref[0]
    ta = ta_ref[0]
    atb = jnp.broadcast_to(att_ref[0], (GH, HID))
    fi = jnp.concatenate([va, ta, atb], axis=-1)
    g = _gelu(_ln(jnp.dot(fi, wg1_ref[...], preferred_element_type=F32)
                  + bg1_ref[...], gg_ref[...], betag_ref[...]))
    s = jnp.dot(g, wg2_ref[...], preferred_element_type=F32) + bg2_ref[...]
    lane = lax.broadcasted_iota(I32, s.shape, 1)
    s = jnp.where(lane < 3, s, -1e30)
    s = s - jnp.max(s, axis=-1, keepdims=True)
    e = jnp.exp(s)
    gw = e / jnp.sum(e, axis=-1, keepdims=True)
    out_ref[0] = (gw[:, 0:1] * va + gw[:, 1:2] * ta + gw[:, 2:3] * atb)


def _tc_gate(va, ta, att83, wg1, bg1, gg, betag, wg2p, bg2p):
    return pl.pallas_call(
        _tc_gate_body,
        grid=(8,),
        in_specs=[
            pl.BlockSpec((1, GH, HID), lambda b: (b, 0, 0)),
            pl.BlockSpec((1, GH, HID), lambda b: (b, 0, 0)),
            pl.BlockSpec((1, 1, HID), lambda b: (b, 0, 0)),
            _full((3 * HID, HID)), _full((1, HID)), _full((1, HID)),
            _full((1, HID)), _full((HID, 128)), _full((1, 128)),
        ],
        out_specs=pl.BlockSpec((1, GH, HID), lambda b: (b, 0, 0)),
        out_shape=jax.ShapeDtypeStruct((8, GH, HID), F32),
    )(va, ta, att83, wg1, bg1, gg, betag, wg2p, bg2p)


def _splat_i32(v):
    return jnp.zeros((16,), I32) + v


def _lanes():
    return lax.iota(I32, 16)


def _zero_acc(acc):
    lanes = _lanes()

    def body(i, _):
        row = _splat_i32(i // 16)
        col = _splat_i32((i % 16) * 16) + lanes
        plsc.store_scatter(acc, [row, col], jnp.zeros((16,), F32))
        return 0

    lax.fori_loop(0, ACC_ROWS * 16, body, 0)


def _alpha_pass(bdstl, bw, den_v):
    lanes = _lanes()
    for i in range(RNG // 16):
        d = den_v[pl.ds(i * 16, 16)]
        den_v[pl.ds(i * 16, 16)] = 1.0 / (d + 1e-16)

    def abody(i, _):
        idx = _splat_i32(i * 16) + lanes
        dl = plsc.load_gather(bdstl, [idx])
        dlc = jnp.minimum(dl, RNG - 1)
        iv = plsc.load_gather(den_v, [dlc])
        w = plsc.load_gather(bw, [idx])
        plsc.store_scatter(bw, [idx], w * iv)
        return 0

    lax.fori_loop(0, BINSZ // 16, abody, 0)


def _accumulate_sub(hp_ref, bsrc, bdstl, bw, acc, stg, sem0, sem1, sub):
    lanes = _lanes()
    base = sub * CAPA2

    def start(g, b, sem):
        pltpu.make_async_copy(
            hp_ref.at[bsrc.at[pl.ds(base + g * GRP, GRP)]],
            stg.at[b], sem).start()

    def wait(b, sem):
        pltpu.make_async_copy(
            hp_ref.at[bsrc.at[pl.ds(0, GRP)]], stg.at[b], sem).wait()

    def process(g, b):
        def ebody(e, _):
            sl = _splat_i32(base + g * GRP) + e
            av = plsc.load_gather(bw, [sl])
            dl = plsc.load_gather(bdstl, [sl])
            rl = jnp.minimum(dl - sub * HRNG, HRNG)
            rl2 = rl + 168
            for t in range(16):
                col = _splat_i32(t * 16) + lanes
                v = stg[b, e, pl.ds(t * 16, 16)]
                lov = plsc.bitcast(v << 16, F32)
                hiv = plsc.bitcast(v & _splat_i32(-65536), F32)
                plsc.addupdate_scatter(acc, [rl, col], lov * av)
                plsc.addupdate_scatter(acc, [rl2, col], hiv * av)
            return 0

        lax.fori_loop(0, GRP, ebody, 0)

    start(0, 0, sem0)

    def body(i, _):
        g0 = i * 2
        start(g0 + 1, 1, sem1)
        wait(0, sem0)
        process(g0, 0)
        start(g0 + 2, 0, sem0)
        wait(1, sem1)
        process(g0 + 1, 1)
        return 0

    lax.fori_loop(0, NG2 // 2, body, 0)
    wait(0, sem0)


def _writeout(acc, agg_h, lo, sub):
    rows = lo + sub * HRNG
    pltpu.sync_copy(acc.at[pl.ds(0, HRNG)],
                    agg_h.at[pl.ds(rows, HRNG), pl.ds(0, HID)])
    pltpu.sync_copy(acc.at[pl.ds(168, HRNG)],
                    agg_h.at[pl.ds(rows, HRNG), pl.ds(HID, HID)])


def _sc_common_tail(hp, agg_h, bsrc, bdstl, bw, den_v, acc, stg,
                    sem0, sem1, lo):
    _alpha_pass(bdstl, bw, den_v)
    for sub in (0, 1):
        _zero_acc(acc)
        _accumulate_sub(hp, bsrc, bdstl, bw, acc, stg, sem0, sem1, sub)
        _writeout(acc, agg_h, lo, sub)


def _make_sc_layer0():
    mesh = plsc.VectorSubcoreMesh(core_axis_name="c", subcore_axis_name="s")

    @functools.partial(
        pl.kernel,
        mesh=mesh,
        compiler_params=pltpu.CompilerParams(needs_layout_passes=False),
        out_type=[
            jax.ShapeDtypeStruct((NP_, GH), F32),
            jax.ShapeDtypeStruct((NW, BINSZ), I32),
            jax.ShapeDtypeStruct((NW, BINSZ), I32),
            jax.ShapeDtypeStruct((NW, 16), I32),
        ],
        scratch_types=[
            pltpu.VMEM((ACC_ROWS, HID), F32),
            pltpu.VMEM((NP_,), F32),
            pltpu.VMEM((RNG,), F32),
            pltpu.VMEM((RNG,), F32),
            pltpu.VMEM((BINSZ,), I32),
            pltpu.VMEM((BINSZ,), I32),
            pltpu.VMEM((BINSZ,), F32),
            pltpu.VMEM((ECHUNK,), I32),
            pltpu.VMEM((ECHUNK,), I32),
            pltpu.VMEM((2, GRP, HID), I32),
            pltpu.VMEM((16,), I32),
            pltpu.SemaphoreType.DMA,
            pltpu.SemaphoreType.DMA,
        ],
    )
    def sc0(hp, asrc_h, adst_h, src_h, dst_h,
            agg_h, bsrc_h, bdstl_h, cnt_h,
            acc, asrc_v, adst_v, den_v, bsrc, bdstl, bw, es, ed, stg, cbuf,
            sem0, sem1):
        c = lax.axis_index("c")
        s = lax.axis_index("s")
        wid = s * 2 + c
        lo = wid * RNG
        lanes = _lanes()

        def init_bins(i, _):
            idx = _splat_i32(i * 16) + lanes
            plsc.store_scatter(bsrc, [idx], jnp.zeros((16,), I32))
            plsc.store_scatter(bdstl, [idx], _splat_i32(RNG))
            plsc.store_scatter(bw, [idx], jnp.zeros((16,), F32))
            return 0

        lax.fori_loop(0, BINSZ // 16, init_bins, 0)
        for i in range(RNG // 16):
            den_v[pl.ds(i * 16, 16)] = jnp.zeros((16,), F32)

        pltpu.sync_copy(asrc_h, asrc_v)
        pltpu.sync_copy(adst_h.at[pl.ds(lo, RNG)], adst_v)

        def chunk(ch, cnts):
            pltpu.sync_copy(src_h.at[pl.ds(ch * ECHUNK, ECHUNK)], es)
            pltpu.sync_copy(dst_h.at[pl.ds(ch * ECHUNK, ECHUNK)], ed)

            def inner(k, cnts):
                c0, c1 = cnts
                idx = _splat_i32(k * 16) + lanes
                sv = plsc.load_gather(es, [idx])
                dv = plsc.load_gather(ed, [idx])
                m = (dv >= lo) & (dv < lo + RNG)
                dl = jnp.clip(dv - lo, 0, RNG - 1)
                av = plsc.load_gather(asrc_v, [sv])
                bv = plsc.load_gather(adst_v, [dl])
                x = av + bv
                e = jnp.where(x >= 0.0, x, 0.2 * x)
                w = jnp.where(m, jnp.exp(e), 0.0)
                plsc.addupdate_scatter(den_v, [dl], w, mask=m)
                m0 = m & (dl < HRNG)
                m1 = m & (dl >= HRNG)
                inc0 = plsc.cumsum(m0.astype(I32))
                pos0 = jnp.minimum(c0 + inc0 - 1, CAP2 - 1)
                plsc.store_scatter(bsrc, [pos0], sv, mask=m0)
                plsc.store_scatter(bdstl, [pos0], dl, mask=m0)
                plsc.store_scatter(bw, [pos0], w, mask=m0)
                inc1 = plsc.cumsum(m1.astype(I32))
                pos1 = jnp.minimum(c1 + inc1 - 1, CAP2 - 1) + CAPA2
                plsc.store_scatter(bsrc, [pos1], sv, mask=m1)
                plsc.store_scatter(bdstl, [pos1], dl, mask=m1)
                plsc.store_scatter(bw, [pos1], w, mask=m1)
                return (c0 + plsc.all_reduce_population_count(m0),
                        c1 + plsc.all_reduce_population_count(m1))

            return lax.fori_loop(0, ECHUNK // 16, inner, cnts)

        z = jnp.zeros((16,), I32)
        c0, c1 = lax.fori_loop(0, NCHUNK, chunk, (z, z))

        cbuf[...] = c0 + c1
        pltpu.sync_copy(bsrc, bsrc_h.at[wid])
        pltpu.sync_copy(bdstl, bdstl_h.at[wid])
        pltpu.sync_copy(cbuf, cnt_h.at[wid])

        _sc_common_tail(hp, agg_h, bsrc, bdstl, bw, den_v, acc, stg,
                        sem0, sem1, lo)

    return sc0


def _make_sc_layer_n():
    mesh = plsc.VectorSubcoreMesh(core_axis_name="c", subcore_axis_name="s")

    @functools.partial(
        pl.kernel,
        mesh=mesh,
        compiler_params=pltpu.CompilerParams(needs_layout_passes=False),
        out_type=jax.ShapeDtypeStruct((NP_, GH), F32),
        scratch_types=[
            pltpu.VMEM((ACC_ROWS, HID), F32),
            pltpu.VMEM((NP_,), F32),
            pltpu.VMEM((RNG,), F32),
            pltpu.VMEM((RNG,), F32),
            pltpu.VMEM((BINSZ,), I32),
            pltpu.VMEM((BINSZ,), I32),
            pltpu.VMEM((BINSZ,), F32),
            pltpu.VMEM((2, GRP, HID), I32),
            pltpu.SemaphoreType.DMA,
            pltpu.SemaphoreType.DMA,
        ],
    )
    def scn(hp, asrc_h, adst_h, bsrc_h, bdstl_h,
            agg_h,
            acc, asrc_v, adst_v, den_v, bsrc, bdstl, bw, stg,
            sem0, sem1):
        c = lax.axis_index("c")
        s = lax.axis_index("s")
        wid = s * 2 + c
        lo = wid * RNG
        lanes = _lanes()

        for i in range(RNG // 16):
            den_v[pl.ds(i * 16, 16)] = jnp.zeros((16,), F32)

        pltpu.sync_copy(asrc_h, asrc_v)
        pltpu.sync_copy(adst_h.at[pl.ds(lo, RNG)], adst_v)
        pltpu.sync_copy(bsrc_h.at[wid], bsrc)
        pltpu.sync_copy(bdstl_h.at[wid], bdstl)

        def wbody(i, _):
            idx = _splat_i32(i * 16) + lanes
            sv = plsc.load_gather(bsrc, [idx])
            dl = plsc.load_gather(bdstl, [idx])
            valid = dl < RNG
            dlc = jnp.minimum(dl, RNG - 1)
            av = plsc.load_gather(asrc_v, [sv])
            bv = plsc.load_gather(adst_v, [dlc])
            x = av + bv
            e = jnp.where(x >= 0.0, x, 0.2 * x)
            w = jnp.where(valid, jnp.exp(e), 0.0)
            plsc.store_scatter(bw, [idx], w)
            plsc.addupdate_scatter(den_v, [dlc], w, mask=valid)
            return 0

        lax.fori_loop(0, BINSZ // 16, wbody, 0)

        _sc_common_tail(hp, agg_h, bsrc, bdstl, bw, den_v, acc, stg,
                        sem0, sem1, lo)

    return scn


_sc_layer0 = _make_sc_layer0()
_sc_layer_n = _make_sc_layer_n()


def kernel(visual_features, text_features, graph_nodes, params, edge_index,
           batch_idx):
    p = params
    B = visual_features.shape[0]

    xp = jnp.concatenate(
        [graph_nodes, jnp.zeros((NP_ - N_REAL, HID), F32)], axis=0)
    loops = jnp.arange(N_REAL, dtype=I32)
    src = jnp.concatenate([edge_index[0].astype(I32), loops,
                           jnp.zeros((ECHUNK * NCHUNK - E_REAL,), I32)])
    dst = jnp.concatenate([edge_index[1].astype(I32), loops,
                           jnp.full((ECHUNK * NCHUNK - E_REAL,), NP_, I32)])
    bidx8 = jnp.broadcast_to(
        jnp.concatenate([batch_idx.astype(I32),
                         jnp.full((NP_ - N_REAL,), 8, I32)])[:, None],
        (NP_, 8))
    row = lambda v: v.reshape(1, -1)
    mat128 = lambda v: jnp.broadcast_to(v.reshape(-1, 1), (v.shape[-1], 128))
    ones = jnp.ones((NP_, 128), F32)
    wg2p = jnp.concatenate([p['Wg2'], jnp.zeros((HID, 125), F32)], axis=1)
    bg2p = jnp.concatenate([p['bg2'], jnp.zeros((125,), F32)]).reshape(1, -1)
    va = jnp.concatenate(
        [visual_features,
         jnp.zeros((B, GH - visual_features.shape[1], HID), F32)], axis=1)

    hp, asr, adr = _tc_head_a(
        xp, p['Wn'], row(p['bn']), row(p['gn']), row(p['betan']), p['W0'],
        mat128(p['as0'][0, 0]), mat128(p['ad0'][0, 0]))
    agg, bsrc_h, bdstl_h, _cnt = _sc_layer0(
        hp, asr[:, 0], adr[:, 0], src, dst)

    for i in (1, 2):
        hp, asr, adr = _tc_head_b(
            agg, row(p['b%d' % (i - 1)]), p['W%d' % i],
            mat128(p['as%d' % i][0, 0]), mat128(p['ad%d' % i][0, 0]))
        agg = _sc_layer_n(hp, asr[:, 0], adr[:, 0], bsrc_h, bdstl_h)

    att8 = _tc_pool(agg, bidx8, row(p['b2']), p['Wop'], row(p['bop']),
                    row(p['gop']), row(p['betaop']), p['Wv'], row(p['bv']),
                    p['Wmo'], row(p['bmo']), ones)

    out = _tc_gate(va, text_features, att8.reshape(8, 1, HID),
                   p['Wg1'], row(p['bg1']), row(p['gg']), row(p['betag']),
                   wg2p, bg2p)
    return out


# speedup vs baseline: 3.4554x; 2.1038x over previous
import functools

import jax
import jax.numpy as jnp
import numpy as np
from jax import lax
from jax.experimental import pallas as pl
from jax.experimental.pallas import tpu as pltpu
from jax.experimental.pallas import tpu_sc as plsc

HID = 256
GH = 512
LN_EPS = 1e-5
N_REAL = 10000
NP_ = 10240
RNG = 320
HRNG = 160
NW = 32
E_REAL = 170000
ECHUNK = 1024
NCHUNK = 168
CAP2 = 3136
GRP = 16
CAPA2 = CAP2 + GRP
BINSZ = 2 * CAPA2
NG2 = CAP2 // GRP
ACC_ROWS = 336
F32 = jnp.float32
I32 = jnp.int32
U32 = jnp.uint32


def _ln(x, g, b):
    m = x.mean(-1, keepdims=True)
    v = ((x - m) ** 2).mean(-1, keepdims=True)
    return (x - m) / jnp.sqrt(v + LN_EPS) * g + b


def _gelu(x):
    return x * 0.5 * (1.0 + lax.erf(x * np.float32(1.0 / np.sqrt(2.0))))


def _pack_bf16(h):
    u = lax.bitcast_convert_type(h, U32)
    b = (u + (((u >> 16) & 1) + 0x7FFF)) >> 16
    lo = b[:, :HID]
    hi = b[:, HID:]
    return lax.bitcast_convert_type(lo | (hi << 16), I32)


def _tc_head_a_body(x_ref, wn_ref, bn_ref, gn_ref, betan_ref, w_ref, am_ref,
                    ad_ref, hp_ref, as_ref, adr_ref):
    x = x_ref[...]
    t = jnp.dot(x, wn_ref[...], preferred_element_type=F32) + bn_ref[...]
    nf = _gelu(_ln(t, gn_ref[...], betan_ref[...]))
    h = jnp.dot(nf, w_ref[...], preferred_element_type=F32)
    hp_ref[...] = _pack_bf16(h)
    as_ref[...] = jnp.dot(h, am_ref[...], preferred_element_type=F32)
    adr_ref[...] = jnp.dot(h, ad_ref[...], preferred_element_type=F32)


def _tc_head_b_body(agg_ref, bprev_ref, w_ref, am_ref, ad_ref,
                    hp_ref, as_ref, adr_ref):
    nf = jnp.maximum(agg_ref[...] + bprev_ref[...], 0.0)
    h = jnp.dot(nf, w_ref[...], preferred_element_type=F32)
    hp_ref[...] = _pack_bf16(h)
    as_ref[...] = jnp.dot(h, am_ref[...], preferred_element_type=F32)
    adr_ref[...] = jnp.dot(h, ad_ref[...], preferred_element_type=F32)


_ROWS_BLK = 256
_NBLK = NP_ // _ROWS_BLK


def _full(shape):
    return pl.BlockSpec(shape, lambda *_: tuple(0 for _ in shape))


def _tc_head_a(xp, wn, bn, gn, betan, w0, am, ad):
    return pl.pallas_call(
        _tc_head_a_body,
        grid=(_NBLK,),
        in_specs=[
            pl.BlockSpec((_ROWS_BLK, HID), lambda i: (i, 0)),
            _full((HID, GH)), _full((1, GH)), _full((1, GH)), _full((1, GH)),
            _full((GH, GH)), _full((GH, 128)), _full((GH, 128)),
        ],
        out_specs=[
            pl.BlockSpec((_ROWS_BLK, HID), lambda i: (i, 0)),
            pl.BlockSpec((_ROWS_BLK, 128), lambda i: (i, 0)),
            pl.BlockSpec((_ROWS_BLK, 128), lambda i: (i, 0)),
        ],
        out_shape=[
            jax.ShapeDtypeStruct((NP_, HID), I32),
            jax.ShapeDtypeStruct((NP_, 128), F32),
            jax.ShapeDtypeStruct((NP_, 128), F32),
        ],
    )(xp, wn, bn, gn, betan, w0, am, ad)


def _tc_head_b(agg, bprev, w, am, ad):
    return pl.pallas_call(
        _tc_head_b_body,
        grid=(_NBLK,),
        in_specs=[
            pl.BlockSpec((_ROWS_BLK, GH), lambda i: (i, 0)),
            _full((1, GH)),
            _full((GH, GH)), _full((GH, 128)), _full((GH, 128)),
        ],
        out_specs=[
            pl.BlockSpec((_ROWS_BLK, HID), lambda i: (i, 0)),
            pl.BlockSpec((_ROWS_BLK, 128), lambda i: (i, 0)),
            pl.BlockSpec((_ROWS_BLK, 128), lambda i: (i, 0)),
        ],
        out_shape=[
            jax.ShapeDtypeStruct((NP_, HID), I32),
            jax.ShapeDtypeStruct((NP_, 128), F32),
            jax.ShapeDtypeStruct((NP_, 128), F32),
        ],
    )(agg, bprev, w, am, ad)


def _tc_pool_body(agg_ref, bidx_ref, b2_ref, wop_ref, bop_ref, gop_ref,
                  betaop_ref, wv_ref, bv_ref, wmo_ref, bmo_ref, ones_ref,
                  att_ref):
    nf = agg_ref[...]
    bi = bidx_ref[...]
    p = (bi == lax.broadcasted_iota(I32, (NP_, 8), 1)).astype(F32)
    pool = lax.dot_general(p, nf, (((0,), (0,)), ((), ())),
                           preferred_element_type=F32)
    cnt = lax.dot_general(p, ones_ref[...], (((0,), (0,)), ((), ())),
                          preferred_element_type=F32)
    gf = pool / jnp.maximum(cnt[:, :1], 1.0) + b2_ref[...]
    gf = _ln(jnp.dot(gf, wop_ref[...], preferred_element_type=F32)
             + bop_ref[...], gop_ref[...], betaop_ref[...])
    att = jnp.dot(gf, wv_ref[...], preferred_element_type=F32) + bv_ref[...]
    att = jnp.dot(att, wmo_ref[...], preferred_element_type=F32) + bmo_ref[...]
    att_ref[...] = att


def _tc_pool(agg2, bidx8, b2, wop, bop, gop, betaop, wv, bv, wmo, bmo, ones):
    return pl.pallas_call(
        _tc_pool_body,
        in_specs=[
            _full((NP_, GH)), _full((NP_, 8)), _full((1, GH)),
            _full((GH, HID)), _full((1, HID)), _full((1, HID)), _full((1, HID)),
            _full((HID, HID)), _full((1, HID)), _full((HID, HID)),
            _full((1, HID)), _full((NP_, 128)),
        ],
        out_specs=_full((8, HID)),
        out_shape=jax.ShapeDtypeStruct((8, HID), F32),
    )(agg2, bidx8, b2, wop, bop, gop, betaop, wv, bv, wmo, bmo, ones)


def _tc_gate_body(va_ref, ta_ref, att_ref, wg1_ref, bg1_ref, gg_ref,
                  betag_ref, wg2_ref, bg2_ref, out_ref):
    va = va_ref[0]
    ta = ta_ref[0]
    atb = jnp.broadcast_to(att_ref[0], (GH, HID))
    fi = jnp.concatenate([va, ta, atb], axis=-1)
    g = _gelu(_ln(jnp.dot(fi, wg1_ref[...], preferred_element_type=F32)
                  + bg1_ref[...], gg_ref[...], betag_ref[...]))
    s = jnp.dot(g, wg2_ref[...], preferred_element_type=F32) + bg2_ref[...]
    lane = lax.broadcasted_iota(I32, s.shape, 1)
    s = jnp.where(lane < 3, s, -1e30)
    s = s - jnp.max(s, axis=-1, keepdims=True)
    e = jnp.exp(s)
    gw = e / jnp.sum(e, axis=-1, keepdims=True)
    out_ref[0] = (gw[:, 0:1] * va + gw[:, 1:2] * ta + gw[:, 2:3] * atb)


def _tc_gate(va, ta, att83, wg1, bg1, gg, betag, wg2p, bg2p):
    return pl.pallas_call(
        _tc_gate_body,
        grid=(8,),
        in_specs=[
            pl.BlockSpec((1, GH, HID), lambda b: (b, 0, 0)),
            pl.BlockSpec((1, GH, HID), lambda b: (b, 0, 0)),
            pl.BlockSpec((1, 1, HID), lambda b: (b, 0, 0)),
            _full((3 * HID, HID)), _full((1, HID)), _full((1, HID)),
            _full((1, HID)), _full((HID, 128)), _full((1, 128)),
        ],
        out_specs=pl.BlockSpec((1, GH, HID), lambda b: (b, 0, 0)),
        out_shape=jax.ShapeDtypeStruct((8, GH, HID), F32),
    )(va, ta, att83, wg1, bg1, gg, betag, wg2p, bg2p)


def _splat_i32(v):
    return jnp.zeros((16,), I32) + v


def _lanes():
    return lax.iota(I32, 16)


def _zero_acc(acc):
    lanes = _lanes()

    def body(i, _):
        row = _splat_i32(i // 16)
        col = _splat_i32((i % 16) * 16) + lanes
        plsc.store_scatter(acc, [row, col], jnp.zeros((16,), F32))
        return 0

    lax.fori_loop(0, ACC_ROWS * 16, body, 0)


def _alpha_pass(bdstl, bw, den_v):
    lanes = _lanes()
    for i in range(RNG // 16):
        d = den_v[pl.ds(i * 16, 16)]
        den_v[pl.ds(i * 16, 16)] = 1.0 / (d + 1e-16)

    def abody(i, _):
        idx = _splat_i32(i * 16) + lanes
        dl = plsc.load_gather(bdstl, [idx])
        dlc = jnp.minimum(dl, RNG - 1)
        iv = plsc.load_gather(den_v, [dlc])
        w = plsc.load_gather(bw, [idx])
        plsc.store_scatter(bw, [idx], w * iv)
        return 0

    lax.fori_loop(0, BINSZ // 16, abody, 0)


def _accumulate_sub(hp_ref, bsrc, bdstl, bw, acc, stg, sem0, sem1, sub, npair):
    lanes = _lanes()
    base = sub * CAPA2

    def start(g, b, sem):
        pltpu.make_async_copy(
            hp_ref.at[bsrc.at[pl.ds(base + g * GRP, GRP)]],
            stg.at[b], sem).start()

    def wait(b, sem):
        pltpu.make_async_copy(
            hp_ref.at[bsrc.at[pl.ds(0, GRP)]], stg.at[b], sem).wait()

    def process(g, b):
        def ebody(e, _):
            sl = _splat_i32(base + g * GRP) + e
            av = plsc.load_gather(bw, [sl])
            dl = plsc.load_gather(bdstl, [sl])
            rl = jnp.minimum(dl - sub * HRNG, HRNG)
            rl2 = rl + 168
            for t in range(16):
                col = _splat_i32(t * 16) + lanes
                v = stg[b, e, pl.ds(t * 16, 16)]
                lov = plsc.bitcast(v << 16, F32)
                hiv = plsc.bitcast(v & _splat_i32(-65536), F32)
                plsc.addupdate_scatter(acc, [rl, col], lov * av)
                plsc.addupdate_scatter(acc, [rl2, col], hiv * av)
            return 0

        lax.fori_loop(0, GRP, ebody, 0)

    start(0, 0, sem0)

    def body(i, _):
        g0 = i * 2
        start(g0 + 1, 1, sem1)
        wait(0, sem0)
        process(g0, 0)
        start(g0 + 2, 0, sem0)
        wait(1, sem1)
        process(g0 + 1, 1)
        return 0

    lax.fori_loop(0, npair, body, 0)
    wait(0, sem0)


def _writeout(acc, agg_h, lo, sub):
    rows = lo + sub * HRNG
    pltpu.sync_copy(acc.at[pl.ds(0, HRNG)],
                    agg_h.at[pl.ds(rows, HRNG), pl.ds(0, HID)])
    pltpu.sync_copy(acc.at[pl.ds(168, HRNG)],
                    agg_h.at[pl.ds(rows, HRNG), pl.ds(HID, HID)])


def _sc_common_tail(hp, agg_h, bsrc, bdstl, bw, den_v, acc, stg,
                    sem0, sem1, lo, n0, n1):
    _alpha_pass(bdstl, bw, den_v)
    for sub, n in ((0, n0), (1, n1)):
        _zero_acc(acc)
        ng = jnp.minimum((n + GRP - 1) // GRP, NG2)
        npair = (ng + 1) // 2
        _accumulate_sub(hp, bsrc, bdstl, bw, acc, stg, sem0, sem1, sub, npair)
        _writeout(acc, agg_h, lo, sub)


def _make_sc_layer0():
    mesh = plsc.VectorSubcoreMesh(core_axis_name="c", subcore_axis_name="s")

    @functools.partial(
        pl.kernel,
        mesh=mesh,
        compiler_params=pltpu.CompilerParams(needs_layout_passes=False),
        out_type=[
            jax.ShapeDtypeStruct((NP_, GH), F32),
            jax.ShapeDtypeStruct((NW, BINSZ), I32),
            jax.ShapeDtypeStruct((NW, BINSZ), I32),
            jax.ShapeDtypeStruct((NW, 32), I32),
        ],
        scratch_types=[
            pltpu.VMEM((ACC_ROWS, HID), F32),
            pltpu.VMEM((NP_,), F32),
            pltpu.VMEM((RNG,), F32),
            pltpu.VMEM((RNG,), F32),
            pltpu.VMEM((BINSZ,), I32),
            pltpu.VMEM((BINSZ,), I32),
            pltpu.VMEM((BINSZ,), F32),
            pltpu.VMEM((ECHUNK,), I32),
            pltpu.VMEM((ECHUNK,), I32),
            pltpu.VMEM((2, GRP, HID), I32),
            pltpu.VMEM((32,), I32),
            pltpu.SemaphoreType.DMA,
            pltpu.SemaphoreType.DMA,
        ],
    )
    def sc0(hp, asrc_h, adst_h, src_h, dst_h,
            agg_h, bsrc_h, bdstl_h, cnt_h,
            acc, asrc_v, adst_v, den_v, bsrc, bdstl, bw, es, ed, stg, cbuf,
            sem0, sem1):
        c = lax.axis_index("c")
        s = lax.axis_index("s")
        wid = s * 2 + c
        lo = wid * RNG
        lanes = _lanes()

        def init_bins(i, _):
            idx = _splat_i32(i * 16) + lanes
            plsc.store_scatter(bsrc, [idx], jnp.zeros((16,), I32))
            plsc.store_scatter(bdstl, [idx], _splat_i32(RNG))
            plsc.store_scatter(bw, [idx], jnp.zeros((16,), F32))
            return 0

        lax.fori_loop(0, BINSZ // 16, init_bins, 0)
        for i in range(RNG // 16):
            den_v[pl.ds(i * 16, 16)] = jnp.zeros((16,), F32)

        pltpu.sync_copy(asrc_h, asrc_v)
        pltpu.sync_copy(adst_h.at[pl.ds(lo, RNG)], adst_v)

        def chunk(ch, cnts):
            pltpu.sync_copy(src_h.at[pl.ds(ch * ECHUNK, ECHUNK)], es)
            pltpu.sync_copy(dst_h.at[pl.ds(ch * ECHUNK, ECHUNK)], ed)

            def inner(k, cnts):
                c0, c1 = cnts
                idx = _splat_i32(k * 16) + lanes
                sv = plsc.load_gather(es, [idx])
                dv = plsc.load_gather(ed, [idx])
                m = (dv >= lo) & (dv < lo + RNG)
                dl = jnp.clip(dv - lo, 0, RNG - 1)
                av = plsc.load_gather(asrc_v, [sv])
                bv = plsc.load_gather(adst_v, [dl])
                x = av + bv
                e = jnp.where(x >= 0.0, x, 0.2 * x)
                w = jnp.where(m, jnp.exp(e), 0.0)
                plsc.addupdate_scatter(den_v, [dl], w, mask=m)
                m0 = m & (dl < HRNG)
                m1 = m & (dl >= HRNG)
                inc0 = plsc.cumsum(m0.astype(I32))
                pos0 = jnp.minimum(c0 + inc0 - 1, CAP2 - 1)
                plsc.store_scatter(bsrc, [pos0], sv, mask=m0)
                plsc.store_scatter(bdstl, [pos0], dl, mask=m0)
                plsc.store_scatter(bw, [pos0], w, mask=m0)
                inc1 = plsc.cumsum(m1.astype(I32))
                pos1 = jnp.minimum(c1 + inc1 - 1, CAP2 - 1) + CAPA2
                plsc.store_scatter(bsrc, [pos1], sv, mask=m1)
                plsc.store_scatter(bdstl, [pos1], dl, mask=m1)
                plsc.store_scatter(bw, [pos1], w, mask=m1)
                return (c0 + plsc.all_reduce_population_count(m0),
                        c1 + plsc.all_reduce_population_count(m1))

            return lax.fori_loop(0, ECHUNK // 16, inner, cnts)

        z = jnp.zeros((16,), I32)
        c0, c1 = lax.fori_loop(0, NCHUNK, chunk, (z, z))

        cbuf[pl.ds(0, 16)] = c0
        cbuf[pl.ds(16, 16)] = c1
        pltpu.sync_copy(bsrc, bsrc_h.at[wid])
        pltpu.sync_copy(bdstl, bdstl_h.at[wid])
        pltpu.sync_copy(cbuf, cnt_h.at[wid])
        n0 = c0[0]
        n1 = c1[0]

        _sc_common_tail(hp, agg_h, bsrc, bdstl, bw, den_v, acc, stg,
                        sem0, sem1, lo, n0, n1)

    return sc0


def _make_sc_layer_n():
    mesh = plsc.VectorSubcoreMesh(core_axis_name="c", subcore_axis_name="s")

    @functools.partial(
        pl.kernel,
        mesh=mesh,
        compiler_params=pltpu.CompilerParams(needs_layout_passes=False),
        out_type=jax.ShapeDtypeStruct((NP_, GH), F32),
        scratch_types=[
            pltpu.VMEM((ACC_ROWS, HID), F32),
            pltpu.VMEM((NP_,), F32),
            pltpu.VMEM((RNG,), F32),
            pltpu.VMEM((RNG,), F32),
            pltpu.VMEM((BINSZ,), I32),
            pltpu.VMEM((BINSZ,), I32),
            pltpu.VMEM((BINSZ,), F32),
            pltpu.VMEM((2, GRP, HID), I32),
            pltpu.VMEM((32,), I32),
            pltpu.SemaphoreType.DMA,
            pltpu.SemaphoreType.DMA,
        ],
    )
    def scn(hp, asrc_h, adst_h, bsrc_h, bdstl_h, cnt_h,
            agg_h,
            acc, asrc_v, adst_v, den_v, bsrc, bdstl, bw, stg, cbuf,
            sem0, sem1):
        c = lax.axis_index("c")
        s = lax.axis_index("s")
        wid = s * 2 + c
        lo = wid * RNG
        lanes = _lanes()

        for i in range(RNG // 16):
            den_v[pl.ds(i * 16, 16)] = jnp.zeros((16,), F32)

        pltpu.sync_copy(asrc_h, asrc_v)
        pltpu.sync_copy(adst_h.at[pl.ds(lo, RNG)], adst_v)
        pltpu.sync_copy(bsrc_h.at[wid], bsrc)
        pltpu.sync_copy(bdstl_h.at[wid], bdstl)
        pltpu.sync_copy(cnt_h.at[wid], cbuf)
        n0 = cbuf[pl.ds(0, 16)][0]
        n1 = cbuf[pl.ds(16, 16)][0]

        def wbody(i, _):
            idx = _splat_i32(i * 16) + lanes
            sv = plsc.load_gather(bsrc, [idx])
            dl = plsc.load_gather(bdstl, [idx])
            valid = dl < RNG
            dlc = jnp.minimum(dl, RNG - 1)
            av = plsc.load_gather(asrc_v, [sv])
            bv = plsc.load_gather(adst_v, [dlc])
            x = av + bv
            e = jnp.where(x >= 0.0, x, 0.2 * x)
            w = jnp.where(valid, jnp.exp(e), 0.0)
            plsc.store_scatter(bw, [idx], w)
            plsc.addupdate_scatter(den_v, [dlc], w, mask=valid)
            return 0

        lax.fori_loop(0, BINSZ // 16, wbody, 0)

        _sc_common_tail(hp, agg_h, bsrc, bdstl, bw, den_v, acc, stg,
                        sem0, sem1, lo, n0, n1)

    return scn


_sc_layer0 = _make_sc_layer0()
_sc_layer_n = _make_sc_layer_n()


def kernel(visual_features, text_features, graph_nodes, params, edge_index,
           batch_idx):
    p = params
    B = visual_features.shape[0]

    xp = jnp.concatenate(
        [graph_nodes, jnp.zeros((NP_ - N_REAL, HID), F32)], axis=0)
    loops = jnp.arange(N_REAL, dtype=I32)
    src = jnp.concatenate([edge_index[0].astype(I32), loops,
                           jnp.zeros((ECHUNK * NCHUNK - E_REAL,), I32)])
    dst = jnp.concatenate([edge_index[1].astype(I32), loops,
                           jnp.full((ECHUNK * NCHUNK - E_REAL,), NP_, I32)])
    bidx8 = jnp.broadcast_to(
        jnp.concatenate([batch_idx.astype(I32),
                         jnp.full((NP_ - N_REAL,), 8, I32)])[:, None],
        (NP_, 8))
    row = lambda v: v.reshape(1, -1)
    mat128 = lambda v: jnp.broadcast_to(v.reshape(-1, 1), (v.shape[-1], 128))
    ones = jnp.ones((NP_, 128), F32)
    wg2p = jnp.concatenate([p['Wg2'], jnp.zeros((HID, 125), F32)], axis=1)
    bg2p = jnp.concatenate([p['bg2'], jnp.zeros((125,), F32)]).reshape(1, -1)
    va = jnp.concatenate(
        [visual_features,
         jnp.zeros((B, GH - visual_features.shape[1], HID), F32)], axis=1)

    hp, asr, adr = _tc_head_a(
        xp, p['Wn'], row(p['bn']), row(p['gn']), row(p['betan']), p['W0'],
        mat128(p['as0'][0, 0]), mat128(p['ad0'][0, 0]))
    agg, bsrc_h, bdstl_h, cnt_h = _sc_layer0(
        hp, asr[:, 0], adr[:, 0], src, dst)

    for i in (1, 2):
        hp, asr, adr = _tc_head_b(
            agg, row(p['b%d' % (i - 1)]), p['W%d' % i],
            mat128(p['as%d' % i][0, 0]), mat128(p['ad%d' % i][0, 0]))
        agg = _sc_layer_n(hp, asr[:, 0], adr[:, 0], bsrc_h, bdstl_h, cnt_h)

    att8 = _tc_pool(agg, bidx8, row(p['b2']), p['Wop'], row(p['bop']),
                    row(p['gop']), row(p['betaop']), p['Wv'], row(p['bv']),
                    p['Wmo'], row(p['bmo']), ones)

    out = _tc_gate(va, text_features, att8.reshape(8, 1, HID),
                   p['Wg1'], row(p['bg1']), row(p['gg']), row(p['betag']),
                   wg2p, bg2p)
    return out
